# Initial kernel scaffold; baseline (speedup 1.0000x reference)
import jax
import jax.numpy as jnp
from jax.experimental import pallas as pl


def kernel(node_ids, timestamps, memory, last_update_time, W_time, b_time, W_fusion, b_fusion):
    raise NotImplementedError("write your pallas kernel here")


# speedup vs baseline: 4.5871x; 4.5871x over previous
import functools

import jax
import jax.numpy as jnp
from jax import lax
from jax.experimental import pallas as pl
from jax.experimental.pallas import tpu as pltpu
from jax.experimental.pallas import tpu_sc as plsc

MEM_DIM = 128
TIME_DIM = 32
IDX_CHUNK = 128


def _sc_gather(ids2, memory, lut2):
    n_rows_idx = ids2.shape[0]
    B = n_rows_idx * IDX_CHUNK
    info = plsc.get_sparse_core_info()
    nw = info.num_cores * info.num_subcores
    b_per_w = B // nw
    chunks = b_per_w // IDX_CHUNK
    mesh = plsc.VectorSubcoreMesh(core_axis_name="c", subcore_axis_name="s")

    @functools.partial(
        pl.kernel,
        mesh=mesh,
        out_type=(
            jax.ShapeDtypeStruct((B, MEM_DIM), jnp.float32),
            jax.ShapeDtypeStruct((B,), jnp.float32),
        ),
        scratch_types=[
            pltpu.VMEM((chunks, IDX_CHUNK), jnp.int32),
            pltpu.VMEM((b_per_w, MEM_DIM), jnp.float32),
            pltpu.VMEM((b_per_w,), jnp.float32),
            pltpu.SemaphoreType.DMA,
            pltpu.SemaphoreType.DMA,
        ],
    )
    def k(ids_hbm, mem_hbm, lut_hbm, rows_out, lt_out, idx_v, rows_v, lt_v,
          sem_r, sem_t):
        wid = lax.axis_index("s") * info.num_cores + lax.axis_index("c")
        base = wid * b_per_w
        pltpu.sync_copy(ids_hbm.at[pl.ds(wid * chunks, chunks)], idx_v)
        cps = []
        for c in range(chunks):
            cps.append(pltpu.async_copy(
                mem_hbm.at[idx_v.at[c]],
                rows_v.at[pl.ds(c * IDX_CHUNK, IDX_CHUNK)], sem_r))
            cps.append(pltpu.async_copy(
                lut_hbm.at[idx_v.at[c]],
                lt_v.at[pl.ds(c * IDX_CHUNK, IDX_CHUNK)], sem_t))
        for cp in cps:
            cp.wait()
        pltpu.sync_copy(rows_v, rows_out.at[pl.ds(base, b_per_w)])
        pltpu.sync_copy(lt_v, lt_out.at[pl.ds(base, b_per_w)])

    return k(ids2, memory, lut2)


def _tc_fuse(rows, lt, ts2, W_time, b_time2, W_fusion, b_fusion2):
    B = rows.shape[0]
    BLK = 1024
    grid = B // BLK

    def body(rows_ref, lt_ref, ts_ref, wt_ref, bt_ref, wf_ref, bf_ref,
             out_ref):
        delta = ts_ref[...] - lt_ref[...]
        enc = jnp.tanh(delta * wt_ref[...] + bt_ref[...])
        h = (jnp.dot(rows_ref[...], wf_ref[:MEM_DIM, :],
                     preferred_element_type=jnp.float32)
             + jnp.dot(enc, wf_ref[MEM_DIM:, :],
                       preferred_element_type=jnp.float32)
             + bf_ref[...])
        out_ref[...] = jnp.tanh(h)

    return pl.pallas_call(
        body,
        grid=(grid,),
        in_specs=[
            pl.BlockSpec((BLK, MEM_DIM), lambda i: (i, 0)),
            pl.BlockSpec((BLK, 1), lambda i: (i, 0)),
            pl.BlockSpec((BLK, 1), lambda i: (i, 0)),
            pl.BlockSpec((1, TIME_DIM), lambda i: (0, 0)),
            pl.BlockSpec((1, TIME_DIM), lambda i: (0, 0)),
            pl.BlockSpec((MEM_DIM + TIME_DIM, MEM_DIM), lambda i: (0, 0)),
            pl.BlockSpec((1, MEM_DIM), lambda i: (0, 0)),
        ],
        out_specs=pl.BlockSpec((BLK, MEM_DIM), lambda i: (i, 0)),
        out_shape=jax.ShapeDtypeStruct((B, MEM_DIM), jnp.float32),
    )(rows, lt, ts2, W_time, b_time2, W_fusion, b_fusion2)


def kernel(node_ids, timestamps, memory, last_update_time, W_time, b_time,
           W_fusion, b_fusion):
    B = node_ids.shape[0]
    ids2 = node_ids.astype(jnp.int32).reshape(B // IDX_CHUNK, IDX_CHUNK)
    rows, lt = _sc_gather(ids2, memory, last_update_time)
    return _tc_fuse(rows, lt.reshape(B, 1), timestamps.reshape(B, 1), W_time,
                    b_time.reshape(1, TIME_DIM), W_fusion,
                    b_fusion.reshape(1, MEM_DIM))


# speedup vs baseline: 5.3772x; 1.1722x over previous
import functools

import jax
import jax.numpy as jnp
from jax import lax
from jax.experimental import pallas as pl
from jax.experimental.pallas import tpu as pltpu
from jax.experimental.pallas import tpu_sc as plsc

MEM_DIM = 128
TIME_DIM = 32
IDX_CHUNK = 128


def _sc_gather(ids2, memory, lut, ts2):
    n_rows_idx = ids2.shape[0]
    B = n_rows_idx * IDX_CHUNK
    info = plsc.get_sparse_core_info()
    nw = info.num_cores * info.num_subcores
    b_per_w = B // nw
    chunks = b_per_w // IDX_CHUNK
    mesh = plsc.VectorSubcoreMesh(core_axis_name="c", subcore_axis_name="s")

    @functools.partial(
        pl.kernel,
        mesh=mesh,
        out_type=(
            jax.ShapeDtypeStruct((B, MEM_DIM), jnp.float32),
            jax.ShapeDtypeStruct((n_rows_idx, IDX_CHUNK), jnp.float32),
        ),
        scratch_types=[
            pltpu.VMEM((chunks, IDX_CHUNK), jnp.int32),
            pltpu.VMEM((b_per_w, MEM_DIM), jnp.float32),
            pltpu.VMEM((chunks, IDX_CHUNK), jnp.float32),
            pltpu.VMEM((chunks, IDX_CHUNK), jnp.float32),
            pltpu.VMEM((chunks, IDX_CHUNK), jnp.float32),
            pltpu.SemaphoreType.DMA,
            pltpu.SemaphoreType.DMA,
            pltpu.SemaphoreType.DMA,
        ],
    )
    def k(ids_hbm, mem_hbm, lut_hbm, ts_hbm, rows_out, d_out,
          idx_v, rows_v, lt_v, ts_v, d_v, sem_r, sem_t, sem_w):
        wid = lax.axis_index("s") * info.num_cores + lax.axis_index("c")
        base = wid * b_per_w
        pltpu.sync_copy(ids_hbm.at[pl.ds(wid * chunks, chunks)], idx_v)
        pltpu.sync_copy(ts_hbm.at[pl.ds(wid * chunks, chunks)], ts_v)
        row_cps = []
        lut_cps = []
        for c in range(chunks):
            row_cps.append(pltpu.async_copy(
                mem_hbm.at[idx_v.at[c]],
                rows_v.at[pl.ds(c * IDX_CHUNK, IDX_CHUNK)], sem_r))
            lut_cps.append(pltpu.async_copy(
                lut_hbm.at[idx_v.at[c]], lt_v.at[c], sem_t))
        wr = []
        for c in range(chunks):
            row_cps[c].wait()
            wr.append(pltpu.async_copy(
                rows_v.at[pl.ds(c * IDX_CHUNK, IDX_CHUNK)],
                rows_out.at[pl.ds(base + c * IDX_CHUNK, IDX_CHUNK)], sem_w))
        for cp in lut_cps:
            cp.wait()
        for c in range(chunks):
            for g in range(IDX_CHUNK // 16):
                sl = pl.ds(g * 16, 16)
                d_v[c, sl] = ts_v[c, sl] - lt_v[c, sl]
        pltpu.sync_copy(d_v, d_out.at[pl.ds(wid * chunks, chunks)])
        for cp in wr:
            cp.wait()

    return k(ids2, memory, lut, ts2)


def _tc_fuse(rows, delta2, W_time, b_time2, W_fusion, b_fusion2, eye):
    B = rows.shape[0]
    BLK = 1024
    grid = B // BLK
    sub = BLK // IDX_CHUNK

    def body(rows_ref, d_ref, wt_ref, bt_ref, wf_ref, bf_ref, eye_ref,
             out_ref):
        i = pl.program_id(0)
        cols = [
            lax.dot_general(
                eye_ref[...], d_ref[pl.ds(sub * i + r, 1), :],
                (((1,), (1,)), ((), ())),
                preferred_element_type=jnp.float32)
            for r in range(sub)
        ]
        dcol = jnp.concatenate(cols, axis=0)
        enc = jnp.tanh(dcol * wt_ref[...] + bt_ref[...])
        h = (jnp.dot(rows_ref[...], wf_ref[:MEM_DIM, :],
                     preferred_element_type=jnp.float32)
             + jnp.dot(enc, wf_ref[MEM_DIM:, :],
                       preferred_element_type=jnp.float32)
             + bf_ref[...])
        out_ref[...] = jnp.tanh(h)

    return pl.pallas_call(
        body,
        grid=(grid,),
        in_specs=[
            pl.BlockSpec((BLK, MEM_DIM), lambda i: (i, 0)),
            pl.BlockSpec((B // IDX_CHUNK, IDX_CHUNK), lambda i: (0, 0)),
            pl.BlockSpec((1, TIME_DIM), lambda i: (0, 0)),
            pl.BlockSpec((1, TIME_DIM), lambda i: (0, 0)),
            pl.BlockSpec((MEM_DIM + TIME_DIM, MEM_DIM), lambda i: (0, 0)),
            pl.BlockSpec((1, MEM_DIM), lambda i: (0, 0)),
            pl.BlockSpec((IDX_CHUNK, IDX_CHUNK), lambda i: (0, 0)),
        ],
        out_specs=pl.BlockSpec((BLK, MEM_DIM), lambda i: (i, 0)),
        out_shape=jax.ShapeDtypeStruct((B, MEM_DIM), jnp.float32),
    )(rows, delta2, W_time, b_time2, W_fusion, b_fusion2, eye)


def kernel(node_ids, timestamps, memory, last_update_time, W_time, b_time,
           W_fusion, b_fusion):
    B = node_ids.shape[0]
    ids2 = node_ids.astype(jnp.int32).reshape(B // IDX_CHUNK, IDX_CHUNK)
    ts2 = timestamps.reshape(B // IDX_CHUNK, IDX_CHUNK)
    rows, delta2 = _sc_gather(ids2, memory, last_update_time, ts2)
    eye = jnp.eye(IDX_CHUNK, dtype=jnp.float32)
    return _tc_fuse(rows, delta2, W_time, b_time.reshape(1, TIME_DIM),
                    W_fusion, b_fusion.reshape(1, MEM_DIM), eye)


# speedup vs baseline: 6.0459x; 1.1244x over previous
import functools

import jax
import jax.numpy as jnp
from jax import lax
from jax.experimental import pallas as pl
from jax.experimental.pallas import tpu as pltpu
from jax.experimental.pallas import tpu_sc as plsc

MEM_DIM = 128
TIME_DIM = 32
IDX_CHUNK = 128


def _sc_gather(ids2, memory, lut, ts2):
    n_rows_idx = ids2.shape[0]
    B = n_rows_idx * IDX_CHUNK
    info = plsc.get_sparse_core_info()
    nw = info.num_cores * info.num_subcores
    b_per_w = B // nw
    chunks = b_per_w // IDX_CHUNK
    mesh = plsc.VectorSubcoreMesh(core_axis_name="c", subcore_axis_name="s")

    @functools.partial(
        pl.kernel,
        mesh=mesh,
        out_type=(
            jax.ShapeDtypeStruct((B, MEM_DIM), jnp.float32),
            jax.ShapeDtypeStruct((n_rows_idx, IDX_CHUNK), jnp.float32),
        ),
        scratch_types=[
            pltpu.VMEM((chunks, IDX_CHUNK), jnp.int32),
            pltpu.VMEM((b_per_w, MEM_DIM), jnp.float32),
            pltpu.VMEM((chunks, IDX_CHUNK), jnp.float32),
            pltpu.VMEM((chunks, IDX_CHUNK), jnp.float32),
            pltpu.VMEM((chunks, IDX_CHUNK), jnp.float32),
            pltpu.SemaphoreType.DMA,
            pltpu.SemaphoreType.DMA,
            pltpu.SemaphoreType.DMA,
        ],
    )
    def k(ids_hbm, mem_hbm, lut_hbm, ts_hbm, rows_out, d_out,
          idx_v, rows_v, lt_v, ts_v, d_v, sem_r, sem_t, sem_w):
        wid = lax.axis_index("s") * info.num_cores + lax.axis_index("c")
        base = wid * b_per_w
        pltpu.sync_copy(ids_hbm.at[pl.ds(wid * chunks, chunks)], idx_v)
        pltpu.sync_copy(ts_hbm.at[pl.ds(wid * chunks, chunks)], ts_v)
        row_cps = []
        lut_cps = []
        for c in range(chunks):
            row_cps.append(pltpu.async_copy(
                mem_hbm.at[idx_v.at[c]],
                rows_v.at[pl.ds(c * IDX_CHUNK, IDX_CHUNK)], sem_r))
            lut_cps.append(pltpu.async_copy(
                lut_hbm.at[idx_v.at[c]], lt_v.at[c], sem_t))
        wr = []
        for c in range(chunks):
            row_cps[c].wait()
            wr.append(pltpu.async_copy(
                rows_v.at[pl.ds(c * IDX_CHUNK, IDX_CHUNK)],
                rows_out.at[pl.ds(base + c * IDX_CHUNK, IDX_CHUNK)], sem_w))
        for cp in lut_cps:
            cp.wait()
        for c in range(chunks):
            for g in range(IDX_CHUNK // 16):
                sl = pl.ds(g * 16, 16)
                d_v[c, sl] = ts_v[c, sl] - lt_v[c, sl]
        pltpu.sync_copy(d_v, d_out.at[pl.ds(wid * chunks, chunks)])
        for cp in wr:
            cp.wait()

    return k(ids2, memory, lut, ts2)


def _tc_fuse(rows, delta2, W_time, b_time2, W_fusion, b_fusion2, eye):
    B = rows.shape[0]
    BLK = 2048
    grid = B // BLK
    sub = BLK // IDX_CHUNK

    def body(rows_ref, d_ref, wt_ref, bt_ref, wf_ref, bf_ref, eye_ref,
             out_ref):
        i = pl.program_id(0)
        cols = [
            lax.dot_general(
                eye_ref[...], d_ref[pl.ds(sub * i + r, 1), :],
                (((1,), (1,)), ((), ())),
                preferred_element_type=jnp.float32)
            for r in range(sub)
        ]
        dcol = jnp.concatenate(cols, axis=0)
        enc = jnp.tanh(dcol * wt_ref[...] + bt_ref[...])
        h = (jnp.dot(rows_ref[...], wf_ref[:MEM_DIM, :],
                     preferred_element_type=jnp.float32)
             + jnp.dot(enc, wf_ref[MEM_DIM:, :],
                       preferred_element_type=jnp.float32)
             + bf_ref[...])
        out_ref[...] = jnp.tanh(h)

    return pl.pallas_call(
        body,
        grid=(grid,),
        in_specs=[
            pl.BlockSpec((BLK, MEM_DIM), lambda i: (i, 0)),
            pl.BlockSpec((B // IDX_CHUNK, IDX_CHUNK), lambda i: (0, 0)),
            pl.BlockSpec((1, TIME_DIM), lambda i: (0, 0)),
            pl.BlockSpec((1, TIME_DIM), lambda i: (0, 0)),
            pl.BlockSpec((MEM_DIM + TIME_DIM, MEM_DIM), lambda i: (0, 0)),
            pl.BlockSpec((1, MEM_DIM), lambda i: (0, 0)),
            pl.BlockSpec((IDX_CHUNK, IDX_CHUNK), lambda i: (0, 0)),
        ],
        out_specs=pl.BlockSpec((BLK, MEM_DIM), lambda i: (i, 0)),
        out_shape=jax.ShapeDtypeStruct((B, MEM_DIM), jnp.float32),
    )(rows, delta2, W_time, b_time2, W_fusion, b_fusion2, eye)


def kernel(node_ids, timestamps, memory, last_update_time, W_time, b_time,
           W_fusion, b_fusion):
    B = node_ids.shape[0]
    ids2 = node_ids.astype(jnp.int32).reshape(B // IDX_CHUNK, IDX_CHUNK)
    ts2 = timestamps.reshape(B // IDX_CHUNK, IDX_CHUNK)
    rows, delta2 = _sc_gather(ids2, memory, last_update_time, ts2)
    eye = jnp.eye(IDX_CHUNK, dtype=jnp.float32)
    return _tc_fuse(rows, delta2, W_time, b_time.reshape(1, TIME_DIM),
                    W_fusion, b_fusion.reshape(1, MEM_DIM), eye)


# speedup vs baseline: 6.3519x; 1.0506x over previous
import functools

import jax
import jax.numpy as jnp
from jax import lax
from jax.experimental import pallas as pl
from jax.experimental.pallas import tpu as pltpu
from jax.experimental.pallas import tpu_sc as plsc

MEM_DIM = 128
TIME_DIM = 32
IDX_CHUNK = 128


def _sc_gather(ids2, memory, lut, ts2):
    n_rows_idx = ids2.shape[0]
    B = n_rows_idx * IDX_CHUNK
    info = plsc.get_sparse_core_info()
    nw = info.num_cores * info.num_subcores
    b_per_w = B // nw
    chunks = b_per_w // IDX_CHUNK
    mesh = plsc.VectorSubcoreMesh(core_axis_name="c", subcore_axis_name="s")

    @functools.partial(
        pl.kernel,
        mesh=mesh,
        out_type=(
            jax.ShapeDtypeStruct((B, MEM_DIM), jnp.float32),
            jax.ShapeDtypeStruct((n_rows_idx, IDX_CHUNK), jnp.float32),
        ),
        scratch_types=[
            pltpu.VMEM((chunks, IDX_CHUNK), jnp.int32),
            pltpu.VMEM((b_per_w, MEM_DIM), jnp.float32),
            pltpu.VMEM((chunks, IDX_CHUNK), jnp.float32),
            pltpu.VMEM((chunks, IDX_CHUNK), jnp.float32),
            pltpu.VMEM((chunks, IDX_CHUNK), jnp.float32),
            pltpu.SemaphoreType.DMA,
            pltpu.SemaphoreType.DMA,
            pltpu.SemaphoreType.DMA,
        ],
    )
    def k(ids_hbm, mem_hbm, lut_hbm, ts_hbm, rows_out, d_out,
          idx_v, rows_v, lt_v, ts_v, d_v, sem_r, sem_t, sem_w):
        wid = lax.axis_index("s") * info.num_cores + lax.axis_index("c")
        base = wid * b_per_w
        pltpu.sync_copy(ids_hbm.at[pl.ds(wid * chunks, chunks)], idx_v)
        pltpu.sync_copy(ts_hbm.at[pl.ds(wid * chunks, chunks)], ts_v)
        row_cps = []
        lut_cps = []
        for c in range(chunks):
            row_cps.append(pltpu.async_copy(
                mem_hbm.at[idx_v.at[c]],
                rows_v.at[pl.ds(c * IDX_CHUNK, IDX_CHUNK)], sem_r))
            lut_cps.append(pltpu.async_copy(
                lut_hbm.at[idx_v.at[c]], lt_v.at[c], sem_t))
        wr = []
        for c in range(chunks):
            row_cps[c].wait()
            wr.append(pltpu.async_copy(
                rows_v.at[pl.ds(c * IDX_CHUNK, IDX_CHUNK)],
                rows_out.at[pl.ds(base + c * IDX_CHUNK, IDX_CHUNK)], sem_w))
        for cp in lut_cps:
            cp.wait()
        for c in range(chunks):
            for g in range(IDX_CHUNK // 16):
                sl = pl.ds(g * 16, 16)
                d_v[c, sl] = ts_v[c, sl] - lt_v[c, sl]
        pltpu.sync_copy(d_v, d_out.at[pl.ds(wid * chunks, chunks)])
        for cp in wr:
            cp.wait()

    return k(ids2, memory, lut, ts2)


def _tc_fuse(rows, delta2, W_time, b_time2, W_fusion, b_fusion2, eye):
    B = rows.shape[0]
    BLK = 4096
    grid = B // BLK
    sub = BLK // IDX_CHUNK

    def body(rows_ref, d_ref, wt_ref, bt_ref, wf_ref, bf_ref, eye_ref,
             out_ref):
        i = pl.program_id(0)
        cols = [
            lax.dot_general(
                eye_ref[...], d_ref[pl.ds(sub * i + r, 1), :],
                (((1,), (1,)), ((), ())),
                preferred_element_type=jnp.float32)
            for r in range(sub)
        ]
        dcol = jnp.concatenate(cols, axis=0)
        enc = jnp.tanh(dcol * wt_ref[...] + bt_ref[...])
        h = (jnp.dot(rows_ref[...], wf_ref[:MEM_DIM, :],
                     preferred_element_type=jnp.float32)
             + jnp.dot(enc, wf_ref[MEM_DIM:, :],
                       preferred_element_type=jnp.float32)
             + bf_ref[...])
        out_ref[...] = jnp.tanh(h)

    return pl.pallas_call(
        body,
        grid=(grid,),
        in_specs=[
            pl.BlockSpec((BLK, MEM_DIM), lambda i: (i, 0)),
            pl.BlockSpec((B // IDX_CHUNK, IDX_CHUNK), lambda i: (0, 0)),
            pl.BlockSpec((1, TIME_DIM), lambda i: (0, 0)),
            pl.BlockSpec((1, TIME_DIM), lambda i: (0, 0)),
            pl.BlockSpec((MEM_DIM + TIME_DIM, MEM_DIM), lambda i: (0, 0)),
            pl.BlockSpec((1, MEM_DIM), lambda i: (0, 0)),
            pl.BlockSpec((IDX_CHUNK, IDX_CHUNK), lambda i: (0, 0)),
        ],
        out_specs=pl.BlockSpec((BLK, MEM_DIM), lambda i: (i, 0)),
        out_shape=jax.ShapeDtypeStruct((B, MEM_DIM), jnp.float32),
    )(rows, delta2, W_time, b_time2, W_fusion, b_fusion2, eye)


def kernel(node_ids, timestamps, memory, last_update_time, W_time, b_time,
           W_fusion, b_fusion):
    B = node_ids.shape[0]
    ids2 = node_ids.astype(jnp.int32).reshape(B // IDX_CHUNK, IDX_CHUNK)
    ts2 = timestamps.reshape(B // IDX_CHUNK, IDX_CHUNK)
    rows, delta2 = _sc_gather(ids2, memory, last_update_time, ts2)
    eye = jnp.eye(IDX_CHUNK, dtype=jnp.float32)
    return _tc_fuse(rows, delta2, W_time, b_time.reshape(1, TIME_DIM),
                    W_fusion, b_fusion.reshape(1, MEM_DIM), eye)


# speedup vs baseline: 6.4311x; 1.0125x over previous
import functools

import jax
import jax.numpy as jnp
from jax import lax
from jax.experimental import pallas as pl
from jax.experimental.pallas import tpu as pltpu
from jax.experimental.pallas import tpu_sc as plsc

MEM_DIM = 128
TIME_DIM = 32
IDX_CHUNK = 128


def _sc_gather(ids2, memory, lut, ts2):
    n_rows_idx = ids2.shape[0]
    B = n_rows_idx * IDX_CHUNK
    info = plsc.get_sparse_core_info()
    nw = info.num_cores * info.num_subcores
    b_per_w = B // nw
    chunks = b_per_w // IDX_CHUNK
    mesh = plsc.VectorSubcoreMesh(core_axis_name="c", subcore_axis_name="s")

    @functools.partial(
        pl.kernel,
        mesh=mesh,
        out_type=(
            jax.ShapeDtypeStruct((B, MEM_DIM), jnp.float32),
            jax.ShapeDtypeStruct((n_rows_idx, IDX_CHUNK), jnp.float32),
        ),
        scratch_types=[
            pltpu.VMEM((chunks, IDX_CHUNK), jnp.int32),
            pltpu.VMEM((b_per_w, MEM_DIM), jnp.float32),
            pltpu.VMEM((chunks, IDX_CHUNK), jnp.float32),
            pltpu.VMEM((chunks, IDX_CHUNK), jnp.float32),
            pltpu.VMEM((chunks, IDX_CHUNK), jnp.float32),
            pltpu.SemaphoreType.DMA,
            pltpu.SemaphoreType.DMA,
            pltpu.SemaphoreType.DMA,
        ],
    )
    def k(ids_hbm, mem_hbm, lut_hbm, ts_hbm, rows_out, d_out,
          idx_v, rows_v, lt_v, ts_v, d_v, sem_r, sem_t, sem_w):
        wid = lax.axis_index("s") * info.num_cores + lax.axis_index("c")
        base = wid * b_per_w
        pltpu.sync_copy(ids_hbm.at[pl.ds(wid * chunks, chunks)], idx_v)
        pltpu.sync_copy(ts_hbm.at[pl.ds(wid * chunks, chunks)], ts_v)
        row_cps = []
        lut_cps = []
        for c in range(chunks):
            row_cps.append(pltpu.async_copy(
                mem_hbm.at[idx_v.at[c]],
                rows_v.at[pl.ds(c * IDX_CHUNK, IDX_CHUNK)], sem_r))
            lut_cps.append(pltpu.async_copy(
                lut_hbm.at[idx_v.at[c]], lt_v.at[c], sem_t))
        wr = []
        for c in range(chunks):
            row_cps[c].wait()
            wr.append(pltpu.async_copy(
                rows_v.at[pl.ds(c * IDX_CHUNK, IDX_CHUNK)],
                rows_out.at[pl.ds(base + c * IDX_CHUNK, IDX_CHUNK)], sem_w))
        for cp in lut_cps:
            cp.wait()
        for c in range(chunks):
            for g in range(IDX_CHUNK // 16):
                sl = pl.ds(g * 16, 16)
                d_v[c, sl] = ts_v[c, sl] - lt_v[c, sl]
        pltpu.sync_copy(d_v, d_out.at[pl.ds(wid * chunks, chunks)])
        for cp in wr:
            cp.wait()

    return k(ids2, memory, lut, ts2)


def _tc_fuse(rows, delta2, W_time, b_time2, W_fusion, b_fusion2, eye):
    B = rows.shape[0]
    BLK = 8192
    grid = B // BLK
    sub = BLK // IDX_CHUNK

    def body(rows_ref, d_ref, wt_ref, bt_ref, wf_ref, bf_ref, eye_ref,
             out_ref):
        i = pl.program_id(0)
        cols = [
            lax.dot_general(
                eye_ref[...], d_ref[pl.ds(sub * i + r, 1), :],
                (((1,), (1,)), ((), ())),
                preferred_element_type=jnp.float32)
            for r in range(sub)
        ]
        dcol = jnp.concatenate(cols, axis=0)
        enc = jnp.tanh(dcol * wt_ref[...] + bt_ref[...])
        h = (jnp.dot(rows_ref[...], wf_ref[:MEM_DIM, :],
                     preferred_element_type=jnp.float32)
             + jnp.dot(enc, wf_ref[MEM_DIM:, :],
                       preferred_element_type=jnp.float32)
             + bf_ref[...])
        out_ref[...] = jnp.tanh(h)

    return pl.pallas_call(
        body,
        grid=(grid,),
        in_specs=[
            pl.BlockSpec((BLK, MEM_DIM), lambda i: (i, 0)),
            pl.BlockSpec((B // IDX_CHUNK, IDX_CHUNK), lambda i: (0, 0)),
            pl.BlockSpec((1, TIME_DIM), lambda i: (0, 0)),
            pl.BlockSpec((1, TIME_DIM), lambda i: (0, 0)),
            pl.BlockSpec((MEM_DIM + TIME_DIM, MEM_DIM), lambda i: (0, 0)),
            pl.BlockSpec((1, MEM_DIM), lambda i: (0, 0)),
            pl.BlockSpec((IDX_CHUNK, IDX_CHUNK), lambda i: (0, 0)),
        ],
        out_specs=pl.BlockSpec((BLK, MEM_DIM), lambda i: (i, 0)),
        out_shape=jax.ShapeDtypeStruct((B, MEM_DIM), jnp.float32),
    )(rows, delta2, W_time, b_time2, W_fusion, b_fusion2, eye)


def kernel(node_ids, timestamps, memory, last_update_time, W_time, b_time,
           W_fusion, b_fusion):
    B = node_ids.shape[0]
    ids2 = node_ids.astype(jnp.int32).reshape(B // IDX_CHUNK, IDX_CHUNK)
    ts2 = timestamps.reshape(B // IDX_CHUNK, IDX_CHUNK)
    rows, delta2 = _sc_gather(ids2, memory, last_update_time, ts2)
    eye = jnp.eye(IDX_CHUNK, dtype=jnp.float32)
    return _tc_fuse(rows, delta2, W_time, b_time.reshape(1, TIME_DIM),
                    W_fusion, b_fusion.reshape(1, MEM_DIM), eye)
